# baseline (speedup 1.0000x reference)
import jax
import jax.numpy as jnp
from jax.experimental import pallas as pl
from jax.experimental.pallas import tpu as pltpu

NUM_ROWS = 17
DIM = 10
B = 16384
LEN = 200

BLK_B = 1024
GRID = B // BLK_B


def _lookup_body(tab_ref, idx_ref, out_ref):
    idxb = idx_ref[...]
    ilo = idxb & 7
    ihi = (idxb - 8) & 7
    is_lo = idxb < 8
    is_16 = idxb == 16
    for d in range(DIM):
        a = jnp.take_along_axis(tab_ref[d, 0:8], ilo, axis=0,
                                mode="promise_in_bounds")
        bv = jnp.take_along_axis(tab_ref[d, 8:16], ihi, axis=0,
                                 mode="promise_in_bounds")
        r = jnp.where(is_lo, a, bv)
        out_ref[d, :, :] = jnp.where(is_16, tab_ref[d, 16], r)


@jax.jit
def _lookup(idx_t, tab_lanes):
    return pl.pallas_call(
        _lookup_body,
        out_shape=jax.ShapeDtypeStruct((DIM, LEN, B), jnp.float32),
        grid=(GRID,),
        in_specs=[
            pl.BlockSpec((DIM, NUM_ROWS, BLK_B), lambda i: (0, 0, 0)),
            pl.BlockSpec((LEN, BLK_B), lambda i: (0, i)),
        ],
        out_specs=pl.BlockSpec((DIM, LEN, BLK_B), lambda i: (0, 0, i)),
        compiler_params=pltpu.CompilerParams(
            dimension_semantics=("parallel",)
        ),
    )(tab_lanes, idx_t)


def kernel(list_POSs, table):
    idx_t = list_POSs.astype(jnp.int32).T
    tab_lanes = jnp.broadcast_to(
        table.astype(jnp.float32).T[:, :, None], (DIM, NUM_ROWS, BLK_B)
    )
    out_t = _lookup(idx_t, tab_lanes)
    return jnp.transpose(out_t, (2, 1, 0))
